# baseline (speedup 1.0000x reference)
import functools

import jax
import jax.numpy as jnp
from jax import lax
from jax.experimental import pallas as pl
from jax.experimental.pallas import tpu as pltpu
from jax.experimental.pallas import tpu_sc as plsc

V = 1000001
D = 64
NB = 16384
NT = 50
NC, NS = 2, 16
NW = NC * NS
BPW = NB // NW
CB = 16
NCHUNK = BPW // CB


def _make_kernel():
    mesh = plsc.VectorSubcoreMesh(core_axis_name="c", subcore_axis_name="s")

    @functools.partial(
        pl.kernel,
        mesh=mesh,
        compiler_params=pltpu.CompilerParams(
            needs_layout_passes=False, use_tc_tiling_on_sc=False
        ),
        out_type=jax.ShapeDtypeStruct((NB, NT, D), jnp.float32),
        scratch_types=[
            pltpu.VMEM((BPW, NT), jnp.int32),
            pltpu.VMEM((CB, NT, D), jnp.float32),
            pltpu.VMEM((CB, NT, D), jnp.float32),
            pltpu.SemaphoreType.DMA,
            pltpu.SemaphoreType.DMA,
            pltpu.SemaphoreType.DMA,
            pltpu.SemaphoreType.DMA,
        ],
    )
    def gather_kernel(
        table_hbm, idx_hbm, out_hbm,
        idx_v, rows0, rows1, gsem0, gsem1, osem0, osem1,
    ):
        wid = lax.axis_index("s") * NC + lax.axis_index("c")
        b_base = wid * BPW
        rows = (rows0, rows1)
        gsem = (gsem0, gsem1)
        osem = (osem0, osem1)
        lane = lax.iota(jnp.int32, 16)
        colg = [lane, lane + 16, lane + 32, lane + (NT - 16)]

        pltpu.sync_copy(idx_hbm.at[pl.ds(b_base, BPW)], idx_v)

        def fire_gather(g, p):
            for j in range(CB):
                pltpu.async_copy(
                    table_hbm.at[idx_v.at[g * CB + j]],
                    rows[p].at[j],
                    gsem[p],
                )

        def drain_gather(p):
            pltpu.make_async_copy(
                rows[p], out_hbm.at[pl.ds(0, CB)], gsem[p]
            ).wait()

        def fire_out(g, p):
            pltpu.async_copy(
                rows[p], out_hbm.at[pl.ds(b_base + g * CB, CB)], osem[p]
            )

        def drain_out(p):
            pltpu.make_async_copy(
                rows[p], out_hbm.at[pl.ds(0, CB)], osem[p]
            ).wait()

        def detect_zero(g):
            mn = jnp.full((16,), 1, jnp.int32)
            for j in range(CB):
                row = jnp.full((16,), g * CB + j, jnp.int32)
                for c in colg:
                    mn = jnp.minimum(mn, plsc.load_gather(idx_v, [row, c]))
            nzero = plsc.all_reduce_population_count(mn == 0)
            return nzero[0] > 0

        def fix_zero_rows(g, p):
            def fixone(e, carry):
                j = e // NT
                t = e % NT
                iv = plsc.load_gather(
                    idx_v,
                    [
                        jnp.full((16,), g * CB + j, jnp.int32),
                        jnp.full((16,), t, jnp.int32),
                    ],
                )
                m = jnp.where(iv == 0, jnp.float32(0.0), jnp.float32(1.0))
                jv = jnp.full((16,), j, jnp.int32)
                tv = jnp.full((16,), t, jnp.int32)
                for k in range(D // 16):
                    col = lane + k * 16
                    x = plsc.load_gather(rows[p], [jv, tv, col])
                    plsc.store_scatter(rows[p], [jv, tv, col], x * m)
                return carry

            lax.fori_loop(0, CB * NT, fixone, 0)

        fire_gather(0, 0)

        def outer(k, carry):
            for b in range(2):
                g = k * 2 + b
                nb = 1 - b
                pl.when((g >= 1) & (g + 1 < NCHUNK))(lambda: drain_out(nb))
                pl.when(g + 1 < NCHUNK)(lambda: fire_gather(g + 1, nb))
                has_zero = detect_zero(g)
                drain_gather(b)
                pl.when(has_zero)(lambda: fix_zero_rows(g, b))
                fire_out(g, b)
            return carry

        lax.fori_loop(0, NCHUNK // 2, outer, 0)
        drain_out(0)
        drain_out(1)

    return gather_kernel


@functools.lru_cache(maxsize=1)
def _get_kernel():
    return _make_kernel()


def kernel(ivectors, data):
    return _get_kernel()(ivectors, data.astype(jnp.int32))
